# Initial kernel scaffold; baseline (speedup 1.0000x reference)
import functools
import jax
import jax.numpy as jnp
from jax import lax
from jax.experimental import pallas as pl
from jax.experimental.pallas import tpu as pltpu
from jax.experimental.pallas import tpu_sc as plsc

NN = 10000
EE = 320000
DD = 128
GG = 64
NC = 2
NS = 16
NW = NC * NS
EPW = EE // NW
CH = 128
NFULL = EPW // CH
REM = EPW - NFULL * CH
RPT = NN // NS
BR = 1000
NB = NN // BR

_mesh = plsc.VectorSubcoreMesh(core_axis_name="c", subcore_axis_name="s")


@functools.partial(
    pl.kernel,
    out_type=jax.ShapeDtypeStruct((NC, NN, 16), jnp.float32),
    mesh=_mesh,
    scratch_types=[
        pltpu.VMEM((CH,), jnp.int32),
        pltpu.VMEM((REM,), jnp.int32),
        pltpu.VMEM((CH, 16), jnp.float32),
        pltpu.VMEM((RPT, 16), jnp.float32),
        pltpu.VMEM_SHARED((NN, 16), jnp.float32),
    ],
)
def _deg_kernel(dst_hbm, out_hbm, didx, didx_r, ones_v, zero_v, acc):
    cid = lax.axis_index("c")
    sid = lax.axis_index("s")
    wid = sid * NC + cid

    @pl.loop(0, RPT)
    def _zfill(i):
        zero_v[i] = jnp.zeros((16,), jnp.float32)

    @pl.loop(0, CH)
    def _ofill(i):
        ones_v[i] = jnp.full((16,), 1.0, jnp.float32)

    pltpu.sync_copy(zero_v, acc.at[pl.ds(sid * RPT, RPT), :])
    plsc.subcore_barrier()

    ebase = wid * EPW

    @pl.loop(0, NFULL)
    def _chunk(j):
        base = ebase + j * CH
        pltpu.sync_copy(dst_hbm.at[pl.ds(base, CH)], didx)
        pltpu.sync_copy(ones_v, acc.at[didx], add=True)

    pltpu.sync_copy(dst_hbm.at[pl.ds(ebase + NFULL * CH, REM)], didx_r)
    pltpu.sync_copy(ones_v.at[pl.ds(0, REM), :], acc.at[didx_r], add=True)

    plsc.subcore_barrier()
    pltpu.sync_copy(
        acc.at[pl.ds(sid * RPT, RPT), :],
        out_hbm.at[cid, pl.ds(sid * RPT, RPT), :],
    )


@functools.partial(
    pl.kernel,
    out_type=jax.ShapeDtypeStruct((NC, NN, DD), jnp.float32),
    mesh=_mesh,
    scratch_types=[
        pltpu.VMEM((CH,), jnp.int32),
        pltpu.VMEM((CH,), jnp.int32),
        pltpu.VMEM((REM,), jnp.int32),
        pltpu.VMEM((REM,), jnp.int32),
        pltpu.VMEM((CH, DD), jnp.float32),
        pltpu.VMEM((REM, DD), jnp.float32),
        pltpu.VMEM((RPT // 5, DD), jnp.float32),
        pltpu.VMEM_SHARED((NN, DD), jnp.float32),
        pltpu.SemaphoreType.DMA,
    ],
)
def _agg_kernel(hws_hbm, src_hbm, dst_hbm, out_hbm,
                sidx, didx, sidx_r, didx_r, rows, rows_r, zero_v, acc, sem):
    cid = lax.axis_index("c")
    sid = lax.axis_index("s")
    wid = sid * NC + cid

    @pl.loop(0, RPT // 5)
    def _zrow(i):
        for k in range(DD // 16):
            zero_v[i, pl.ds(k * 16, 16)] = jnp.zeros((16,), jnp.float32)

    for k in range(5):
        pltpu.sync_copy(zero_v, acc.at[pl.ds(sid * RPT + k * (RPT // 5), RPT // 5), :])
    plsc.subcore_barrier()

    ebase = wid * EPW

    @pl.loop(0, NFULL)
    def _chunk(j):
        base = ebase + j * CH
        pltpu.sync_copy(src_hbm.at[pl.ds(base, CH)], sidx)
        pltpu.sync_copy(dst_hbm.at[pl.ds(base, CH)], didx)
        pltpu.async_copy(hws_hbm.at[sidx], rows, sem).wait()
        pltpu.sync_copy(rows, acc.at[didx], add=True)

    rbase = ebase + NFULL * CH
    pltpu.sync_copy(src_hbm.at[pl.ds(rbase, REM)], sidx_r)
    pltpu.sync_copy(dst_hbm.at[pl.ds(rbase, REM)], didx_r)
    pltpu.async_copy(hws_hbm.at[sidx_r], rows_r, sem).wait()
    pltpu.sync_copy(rows_r, acc.at[didx_r], add=True)

    plsc.subcore_barrier()
    pltpu.sync_copy(
        acc.at[pl.ds(sid * RPT, RPT), :],
        out_hbm.at[cid, pl.ds(sid * RPT, RPT), :],
    )


def _first_body(x_ref, w_ref, d0_ref, d1_ref, hws_ref, dinv_ref):
    deg = d0_ref[:, 0:1] + d1_ref[:, 0:1] + 1.0
    dinv = lax.rsqrt(deg)
    dinv_ref[...] = dinv
    hws_ref[...] = (
        jnp.dot(x_ref[...], w_ref[...], preferred_element_type=jnp.float32) * dinv
    )


_first_call = pl.pallas_call(
    _first_body,
    grid=(NB,),
    in_specs=[
        pl.BlockSpec((BR, DD), lambda i: (i, 0)),
        pl.BlockSpec((DD, DD), lambda i: (0, 0)),
        pl.BlockSpec((BR, 16), lambda i: (i, 0)),
        pl.BlockSpec((BR, 16), lambda i: (i, 0)),
    ],
    out_specs=[
        pl.BlockSpec((BR, DD), lambda i: (i, 0)),
        pl.BlockSpec((BR, 1), lambda i: (i, 0)),
    ],
    out_shape=[
        jax.ShapeDtypeStruct((NN, DD), jnp.float32),
        jax.ShapeDtypeStruct((NN, 1), jnp.float32),
    ],
)


def _layer_body(p0_ref, p1_ref, hws_ref, dinv_ref, batch_ref, b_ref, w_ref,
                hwsn_ref, pool_ref):
    i = pl.program_id(0)
    dinv = dinv_ref[...]
    t = (p0_ref[...] + p1_ref[...] + hws_ref[...]) * dinv
    h = jnp.maximum(t + b_ref[...], 0.0)
    oneh = (batch_ref[...] == lax.broadcasted_iota(jnp.int32, (BR, GG), 1))
    oneh = oneh.astype(jnp.float32)
    contrib = lax.dot_general(
        oneh, h, (((0,), (0,)), ((), ())), preferred_element_type=jnp.float32
    )

    @pl.when(i == 0)
    def _():
        pool_ref[...] = jnp.zeros_like(pool_ref)

    pool_ref[...] += contrib
    hwsn_ref[...] = (
        jnp.dot(h, w_ref[...], preferred_element_type=jnp.float32) * dinv
    )


_layer_call = pl.pallas_call(
    _layer_body,
    grid=(NB,),
    in_specs=[
        pl.BlockSpec((BR, DD), lambda i: (i, 0)),
        pl.BlockSpec((BR, DD), lambda i: (i, 0)),
        pl.BlockSpec((BR, DD), lambda i: (i, 0)),
        pl.BlockSpec((BR, 1), lambda i: (i, 0)),
        pl.BlockSpec((BR, 1), lambda i: (i, 0)),
        pl.BlockSpec((1, DD), lambda i: (0, 0)),
        pl.BlockSpec((DD, DD), lambda i: (0, 0)),
    ],
    out_specs=[
        pl.BlockSpec((BR, DD), lambda i: (i, 0)),
        pl.BlockSpec((GG, DD), lambda i: (0, 0)),
    ],
    out_shape=[
        jax.ShapeDtypeStruct((NN, DD), jnp.float32),
        jax.ShapeDtypeStruct((GG, DD), jnp.float32),
    ],
)


def _last_body(p0_ref, p1_ref, hws_ref, dinv_ref, batch_ref, b_ref, pool_ref):
    i = pl.program_id(0)
    t = (p0_ref[...] + p1_ref[...] + hws_ref[...]) * dinv_ref[...]
    h = jnp.maximum(t + b_ref[...], 0.0)
    oneh = (batch_ref[...] == lax.broadcasted_iota(jnp.int32, (BR, GG), 1))
    oneh = oneh.astype(jnp.float32)
    contrib = lax.dot_general(
        oneh, h, (((0,), (0,)), ((), ())), preferred_element_type=jnp.float32
    )

    @pl.when(i == 0)
    def _():
        pool_ref[...] = jnp.zeros_like(pool_ref)

    pool_ref[...] += contrib


_last_call = pl.pallas_call(
    _last_body,
    grid=(NB,),
    in_specs=[
        pl.BlockSpec((BR, DD), lambda i: (i, 0)),
        pl.BlockSpec((BR, DD), lambda i: (i, 0)),
        pl.BlockSpec((BR, DD), lambda i: (i, 0)),
        pl.BlockSpec((BR, 1), lambda i: (i, 0)),
        pl.BlockSpec((BR, 1), lambda i: (i, 0)),
        pl.BlockSpec((1, DD), lambda i: (0, 0)),
    ],
    out_specs=pl.BlockSpec((GG, DD), lambda i: (0, 0)),
    out_shape=jax.ShapeDtypeStruct((GG, DD), jnp.float32),
)


def kernel(x, edge_index, batch, Ws, bs):
    edge_index = edge_index.astype(jnp.int32)
    src = edge_index[0]
    dst = edge_index[1]
    batch2 = batch.astype(jnp.int32).reshape(NN, 1)

    degp = _deg_kernel(dst)
    hws, dinv = _first_call(x, Ws[0], degp[0], degp[1])

    pools = []
    for i in range(5):
        parts = _agg_kernel(hws, src, dst)
        if i < 4:
            hws, pool = _layer_call(
                parts[0], parts[1], hws, dinv, batch2,
                bs[i].reshape(1, DD), Ws[i + 1]
            )
        else:
            pool = _last_call(
                parts[0], parts[1], hws, dinv, batch2, bs[i].reshape(1, DD)
            )
        pools.append(pool)

    return jnp.concatenate(pools, axis=1)


# speedup vs baseline: 5.1475x; 5.1475x over previous
import functools
import jax
import jax.numpy as jnp
from jax import lax
from jax.experimental import pallas as pl
from jax.experimental.pallas import tpu as pltpu
from jax.experimental.pallas import tpu_sc as plsc

NN = 10000
EE = 320000
DD = 128
GG = 64
NC = 2
NS = 16
NW = NC * NS
CH = 128
NNP = 10240
RPT = NNP // NS
ZR = RPT // 5
EPW = 10240
EP = NW * EPW
NCHUNK = EPW // CH
BR = 1000
NB = NN // BR

_mesh = plsc.VectorSubcoreMesh(core_axis_name="c", subcore_axis_name="s")


def _zero_acc(zero_v, acc, sid, sem):
    @pl.loop(0, ZR)
    def _z(i):
        for k in range(DD // 16):
            zero_v[i, pl.ds(k * 16, 16)] = jnp.zeros((16,), jnp.float32)

    for k in range(5):
        pltpu.async_copy(
            zero_v, acc.at[pl.ds(sid * RPT + k * ZR, ZR), :], sem
        ).wait()


@functools.partial(
    pl.kernel,
    out_type=jax.ShapeDtypeStruct((NC, NNP, DD), jnp.float32),
    mesh=_mesh,
    scratch_types=[
        pltpu.VMEM((CH,), jnp.int32),
        pltpu.VMEM((CH, DD), jnp.float32),
        pltpu.VMEM((ZR, DD), jnp.float32),
        pltpu.VMEM_SHARED((NNP, DD), jnp.float32),
        pltpu.SemaphoreType.DMA,
    ],
)
def _deg_kernel(dst_hbm, out_hbm, didx, ones_v, zero_v, acc, sem):
    cid = lax.axis_index("c")
    sid = lax.axis_index("s")
    wid = sid * NC + cid

    @pl.loop(0, CH)
    def _o(i):
        for k in range(DD // 16):
            ones_v[i, pl.ds(k * 16, 16)] = jnp.full((16,), 1.0, jnp.float32)

    _zero_acc(zero_v, acc, sid, sem)
    plsc.subcore_barrier()

    ebase = wid * EPW

    @pl.loop(0, NCHUNK)
    def _chunk(j):
        base = ebase + j * CH
        pltpu.async_copy(dst_hbm.at[pl.ds(base, CH)], didx, sem).wait()
        pltpu.async_copy(ones_v, acc.at[didx], sem, add=True).wait()

    plsc.subcore_barrier()
    pltpu.async_copy(
        acc.at[pl.ds(sid * RPT, RPT), :],
        out_hbm.at[cid, pl.ds(sid * RPT, RPT), :],
        sem,
    ).wait()


@functools.partial(
    pl.kernel,
    out_type=jax.ShapeDtypeStruct((NC, NNP, DD), jnp.float32),
    mesh=_mesh,
    scratch_types=[
        pltpu.VMEM((CH,), jnp.int32),
        pltpu.VMEM((CH,), jnp.int32),
        pltpu.VMEM((CH, DD), jnp.float32),
        pltpu.VMEM((ZR, DD), jnp.float32),
        pltpu.VMEM_SHARED((NNP, DD), jnp.float32),
        pltpu.SemaphoreType.DMA,
    ],
)
def _agg_kernel(hws_hbm, src_hbm, dst_hbm, out_hbm,
                sidx, didx, rows, zero_v, acc, sem):
    cid = lax.axis_index("c")
    sid = lax.axis_index("s")
    wid = sid * NC + cid

    _zero_acc(zero_v, acc, sid, sem)
    plsc.subcore_barrier()

    ebase = wid * EPW

    @pl.loop(0, NCHUNK)
    def _chunk(j):
        base = ebase + j * CH
        pltpu.async_copy(src_hbm.at[pl.ds(base, CH)], sidx, sem).wait()
        pltpu.async_copy(dst_hbm.at[pl.ds(base, CH)], didx, sem).wait()
        pltpu.async_copy(hws_hbm.at[sidx], rows, sem).wait()
        pltpu.async_copy(rows, acc.at[didx], sem, add=True).wait()

    plsc.subcore_barrier()
    pltpu.async_copy(
        acc.at[pl.ds(sid * RPT, RPT), :],
        out_hbm.at[cid, pl.ds(sid * RPT, RPT), :],
        sem,
    ).wait()


def _first_body(x_ref, w_ref, d0_ref, d1_ref, hws_ref, dinv_ref):
    deg = d0_ref[0, :, 0:1] + d1_ref[0, :, 0:1] + 1.0
    dinv = lax.rsqrt(deg)
    dinv_ref[...] = dinv
    hws_ref[...] = (
        jnp.dot(x_ref[...], w_ref[...], preferred_element_type=jnp.float32) * dinv
    )


_first_call = pl.pallas_call(
    _first_body,
    grid=(NB,),
    in_specs=[
        pl.BlockSpec((BR, DD), lambda i: (i, 0)),
        pl.BlockSpec((DD, DD), lambda i: (0, 0)),
        pl.BlockSpec((1, BR, DD), lambda i: (0, i, 0)),
        pl.BlockSpec((1, BR, DD), lambda i: (1, i, 0)),
    ],
    out_specs=[
        pl.BlockSpec((BR, DD), lambda i: (i, 0)),
        pl.BlockSpec((BR, 1), lambda i: (i, 0)),
    ],
    out_shape=[
        jax.ShapeDtypeStruct((NN, DD), jnp.float32),
        jax.ShapeDtypeStruct((NN, 1), jnp.float32),
    ],
)


def _layer_body(p0_ref, p1_ref, hws_ref, dinv_ref, batch_ref, b_ref, w_ref,
                hwsn_ref, pool_ref):
    i = pl.program_id(0)
    dinv = dinv_ref[...]
    t = (p0_ref[0] + p1_ref[0] + hws_ref[...]) * dinv
    h = jnp.maximum(t + b_ref[...], 0.0)
    oneh = (batch_ref[...] == lax.broadcasted_iota(jnp.int32, (BR, GG), 1))
    oneh = oneh.astype(jnp.float32)
    contrib = lax.dot_general(
        oneh, h, (((0,), (0,)), ((), ())), preferred_element_type=jnp.float32
    )

    @pl.when(i == 0)
    def _():
        pool_ref[...] = jnp.zeros_like(pool_ref)

    pool_ref[...] += contrib
    hwsn_ref[...] = (
        jnp.dot(h, w_ref[...], preferred_element_type=jnp.float32) * dinv
    )


_layer_call = pl.pallas_call(
    _layer_body,
    grid=(NB,),
    in_specs=[
        pl.BlockSpec((1, BR, DD), lambda i: (0, i, 0)),
        pl.BlockSpec((1, BR, DD), lambda i: (1, i, 0)),
        pl.BlockSpec((BR, DD), lambda i: (i, 0)),
        pl.BlockSpec((BR, 1), lambda i: (i, 0)),
        pl.BlockSpec((BR, 1), lambda i: (i, 0)),
        pl.BlockSpec((1, DD), lambda i: (0, 0)),
        pl.BlockSpec((DD, DD), lambda i: (0, 0)),
    ],
    out_specs=[
        pl.BlockSpec((BR, DD), lambda i: (i, 0)),
        pl.BlockSpec((GG, DD), lambda i: (0, 0)),
    ],
    out_shape=[
        jax.ShapeDtypeStruct((NN, DD), jnp.float32),
        jax.ShapeDtypeStruct((GG, DD), jnp.float32),
    ],
)


def _last_body(p0_ref, p1_ref, hws_ref, dinv_ref, batch_ref, b_ref, pool_ref):
    i = pl.program_id(0)
    t = (p0_ref[0] + p1_ref[0] + hws_ref[...]) * dinv_ref[...]
    h = jnp.maximum(t + b_ref[...], 0.0)
    oneh = (batch_ref[...] == lax.broadcasted_iota(jnp.int32, (BR, GG), 1))
    oneh = oneh.astype(jnp.float32)
    contrib = lax.dot_general(
        oneh, h, (((0,), (0,)), ((), ())), preferred_element_type=jnp.float32
    )

    @pl.when(i == 0)
    def _():
        pool_ref[...] = jnp.zeros_like(pool_ref)

    pool_ref[...] += contrib


_last_call = pl.pallas_call(
    _last_body,
    grid=(NB,),
    in_specs=[
        pl.BlockSpec((1, BR, DD), lambda i: (0, i, 0)),
        pl.BlockSpec((1, BR, DD), lambda i: (1, i, 0)),
        pl.BlockSpec((BR, DD), lambda i: (i, 0)),
        pl.BlockSpec((BR, 1), lambda i: (i, 0)),
        pl.BlockSpec((BR, 1), lambda i: (i, 0)),
        pl.BlockSpec((1, DD), lambda i: (0, 0)),
    ],
    out_specs=pl.BlockSpec((GG, DD), lambda i: (0, 0)),
    out_shape=jax.ShapeDtypeStruct((GG, DD), jnp.float32),
)


def kernel(x, edge_index, batch, Ws, bs):
    edge_index = edge_index.astype(jnp.int32)
    npad = EP - EE
    src = jnp.concatenate([edge_index[0], jnp.zeros((npad,), jnp.int32)])
    dst = jnp.concatenate([edge_index[1], jnp.full((npad,), NN, jnp.int32)])
    batch2 = batch.astype(jnp.int32).reshape(NN, 1)

    degp = _deg_kernel(dst)
    hws, dinv = _first_call(x, Ws[0], degp, degp)

    pools = []
    for i in range(5):
        parts = _agg_kernel(hws, src, dst)
        if i < 4:
            hws, pool = _layer_call(
                parts, parts, hws, dinv, batch2,
                bs[i].reshape(1, DD), Ws[i + 1]
            )
        else:
            pool = _last_call(
                parts, parts, hws, dinv, batch2, bs[i].reshape(1, DD)
            )
        pools.append(pool)

    return jnp.concatenate(pools, axis=1)


# speedup vs baseline: 6.1849x; 1.2015x over previous
import functools
import jax
import jax.numpy as jnp
from jax import lax
from jax.experimental import pallas as pl
from jax.experimental.pallas import tpu as pltpu
from jax.experimental.pallas import tpu_sc as plsc

NN = 10000
EE = 320000
DD = 128
GG = 64
NC = 2
NS = 16
NW = NC * NS
CH = 128
NNP = 10240
RPT = NNP // NS
ZR = RPT // 5
EPW = 10240
EP = NW * EPW
NCHUNK = EPW // CH
BR = 1000
NB = NN // BR

_mesh = plsc.VectorSubcoreMesh(core_axis_name="c", subcore_axis_name="s")


def _zero_acc(zero_v, acc, sid, sem):
    @pl.loop(0, ZR)
    def _z(i):
        for k in range(DD // 16):
            zero_v[i, pl.ds(k * 16, 16)] = jnp.zeros((16,), jnp.float32)

    for k in range(5):
        pltpu.async_copy(
            zero_v.at[pl.ds(0, ZR), :], acc.at[pl.ds(sid * RPT + k * ZR, ZR), :], sem
        )
    for k in range(5):
        pltpu.make_async_copy(
            zero_v.at[pl.ds(0, ZR), :], acc.at[pl.ds(sid * RPT, ZR), :], sem
        ).wait()


@functools.partial(
    pl.kernel,
    out_type=jax.ShapeDtypeStruct((NC, NNP, DD), jnp.float32),
    mesh=_mesh,
    scratch_types=[
        [pltpu.VMEM((CH,), jnp.int32) for _ in range(4)],
        pltpu.VMEM((CH, DD), jnp.float32),
        pltpu.VMEM_SHARED((NNP, DD), jnp.float32),
        pltpu.SemaphoreType.DMA,
        pltpu.SemaphoreType.DMA,
    ],
)
def _deg_kernel(dst_hbm, out_hbm, dbufs, ones_v, acc, isem, ssem):
    cid = lax.axis_index("c")
    sid = lax.axis_index("s")
    wid = sid * NC + cid
    ebase = wid * EPW

    def start_i(j, buf):
        pltpu.async_copy(dst_hbm.at[pl.ds(ebase + j * CH, CH)], buf, isem)

    def wait_i(buf):
        pltpu.make_async_copy(dst_hbm.at[pl.ds(ebase, CH)], buf, isem).wait()

    def wait_s():
        pltpu.make_async_copy(ones_v, acc.at[dbufs[0]], ssem).wait()

    for m in range(3):
        start_i(m, dbufs[m])
    _zero_acc(ones_v, acc, sid, ssem)

    @pl.loop(0, CH)
    def _o(i):
        for k in range(DD // 16):
            ones_v[i, pl.ds(k * 16, 16)] = jnp.full((16,), 1.0, jnp.float32)

    wait_i(dbufs[0])
    plsc.subcore_barrier()

    Q = NCHUNK // 4

    @pl.loop(0, Q)
    def _chunk(q):
        for m in range(4):
            j = q * 4 + m
            pltpu.async_copy(ones_v, acc.at[dbufs[m]], ssem, add=True)
            if m == 0:
                @pl.when(q > 0)
                def _():
                    wait_s()
            else:
                wait_s()
            if m == 1 or m == 2 or m == 3:
                @pl.when(q < Q - 1)
                def _():
                    start_i(j + 3, dbufs[(m + 3) % 4])
            else:
                start_i(j + 3, dbufs[3])
            if m < 3:
                wait_i(dbufs[m + 1])
            else:
                @pl.when(q < Q - 1)
                def _():
                    wait_i(dbufs[0])

    wait_s()

    plsc.subcore_barrier()
    pltpu.async_copy(
        acc.at[pl.ds(sid * RPT, RPT), :],
        out_hbm.at[cid, pl.ds(sid * RPT, RPT), :],
        ssem,
    ).wait()


@functools.partial(
    pl.kernel,
    out_type=jax.ShapeDtypeStruct((NC, NNP, DD), jnp.float32),
    mesh=_mesh,
    scratch_types=[
        [pltpu.VMEM((CH,), jnp.int32) for _ in range(4)],
        [pltpu.VMEM((CH,), jnp.int32) for _ in range(4)],
        pltpu.VMEM((CH, DD), jnp.float32),
        pltpu.VMEM((CH, DD), jnp.float32),
        pltpu.VMEM_SHARED((NNP, DD), jnp.float32),
        pltpu.SemaphoreType.DMA,
        pltpu.SemaphoreType.DMA,
        pltpu.SemaphoreType.DMA,
    ],
)
def _agg_kernel(hws_hbm, src_hbm, dst_hbm, out_hbm,
                sbufs, dbufs, rows0, rows1, acc,
                isem, gsem, ssem):
    cid = lax.axis_index("c")
    sid = lax.axis_index("s")
    wid = sid * NC + cid
    rbufs = (rows0, rows1)
    ebase = wid * EPW

    def start_i(j, m):
        pltpu.async_copy(src_hbm.at[pl.ds(ebase + j * CH, CH)], sbufs[m], isem)
        pltpu.async_copy(dst_hbm.at[pl.ds(ebase + j * CH, CH)], dbufs[m], isem)

    def wait_i(m):
        pltpu.make_async_copy(src_hbm.at[pl.ds(ebase, CH)], sbufs[m], isem).wait()
        pltpu.make_async_copy(dst_hbm.at[pl.ds(ebase, CH)], dbufs[m], isem).wait()

    def start_g(m, buf):
        pltpu.async_copy(hws_hbm.at[sbufs[m]], buf, gsem)

    def wait_g(buf):
        pltpu.make_async_copy(hws_hbm.at[sbufs[0]], buf, gsem).wait()

    def start_s(m, buf):
        pltpu.async_copy(buf, acc.at[dbufs[m]], ssem, add=True)

    def wait_s(buf):
        pltpu.make_async_copy(buf, acc.at[dbufs[0]], ssem).wait()

    for m in range(3):
        start_i(m, m)
    _zero_acc(rows0, acc, sid, gsem)
    wait_i(0)
    plsc.subcore_barrier()
    start_g(0, rows0)

    Q = NCHUNK // 4

    @pl.loop(0, Q)
    def _chunk(q):
        for m in range(4):
            j = q * 4 + m
            cur = rbufs[m % 2]
            oth = rbufs[1 - m % 2]

            wait_g(cur)
            start_s(m, cur)
            if m == 0:
                @pl.when(q > 0)
                def _():
                    wait_s(oth)
            else:
                wait_s(oth)
            if m < 3:
                wait_i(m + 1)
                start_g(m + 1, oth)
            else:
                @pl.when(q < Q - 1)
                def _():
                    wait_i(0)
                    start_g(0, oth)
            if m == 0:
                start_i(j + 3, 3)
            else:
                @pl.when(q < Q - 1)
                def _():
                    start_i(j + 3, (m + 3) % 4)

    wait_s(rows0)

    plsc.subcore_barrier()
    pltpu.async_copy(
        acc.at[pl.ds(sid * RPT, RPT), :],
        out_hbm.at[cid, pl.ds(sid * RPT, RPT), :],
        ssem,
    ).wait()


def _first_body(x_ref, w_ref, d0_ref, d1_ref, hws_ref, dinv_ref):
    deg = d0_ref[0, :, 0:1] + d1_ref[0, :, 0:1] + 1.0
    dinv = lax.rsqrt(deg)
    dinv_ref[...] = dinv
    hws_ref[...] = (
        jnp.dot(x_ref[...], w_ref[...], preferred_element_type=jnp.float32) * dinv
    )


_first_call = pl.pallas_call(
    _first_body,
    grid=(NB,),
    in_specs=[
        pl.BlockSpec((BR, DD), lambda i: (i, 0)),
        pl.BlockSpec((DD, DD), lambda i: (0, 0)),
        pl.BlockSpec((1, BR, DD), lambda i: (0, i, 0)),
        pl.BlockSpec((1, BR, DD), lambda i: (1, i, 0)),
    ],
    out_specs=[
        pl.BlockSpec((BR, DD), lambda i: (i, 0)),
        pl.BlockSpec((BR, 1), lambda i: (i, 0)),
    ],
    out_shape=[
        jax.ShapeDtypeStruct((NN, DD), jnp.float32),
        jax.ShapeDtypeStruct((NN, 1), jnp.float32),
    ],
)


def _layer_body(p0_ref, p1_ref, hws_ref, dinv_ref, batch_ref, b_ref, w_ref,
                hwsn_ref, pool_ref):
    i = pl.program_id(0)
    dinv = dinv_ref[...]
    t = (p0_ref[0] + p1_ref[0] + hws_ref[...]) * dinv
    h = jnp.maximum(t + b_ref[...], 0.0)
    oneh = (batch_ref[...] == lax.broadcasted_iota(jnp.int32, (BR, GG), 1))
    oneh = oneh.astype(jnp.float32)
    contrib = lax.dot_general(
        oneh, h, (((0,), (0,)), ((), ())), preferred_element_type=jnp.float32
    )

    @pl.when(i == 0)
    def _():
        pool_ref[...] = jnp.zeros_like(pool_ref)

    pool_ref[...] += contrib
    hwsn_ref[...] = (
        jnp.dot(h, w_ref[...], preferred_element_type=jnp.float32) * dinv
    )


_layer_call = pl.pallas_call(
    _layer_body,
    grid=(NB,),
    in_specs=[
        pl.BlockSpec((1, BR, DD), lambda i: (0, i, 0)),
        pl.BlockSpec((1, BR, DD), lambda i: (1, i, 0)),
        pl.BlockSpec((BR, DD), lambda i: (i, 0)),
        pl.BlockSpec((BR, 1), lambda i: (i, 0)),
        pl.BlockSpec((BR, 1), lambda i: (i, 0)),
        pl.BlockSpec((1, DD), lambda i: (0, 0)),
        pl.BlockSpec((DD, DD), lambda i: (0, 0)),
    ],
    out_specs=[
        pl.BlockSpec((BR, DD), lambda i: (i, 0)),
        pl.BlockSpec((GG, DD), lambda i: (0, 0)),
    ],
    out_shape=[
        jax.ShapeDtypeStruct((NN, DD), jnp.float32),
        jax.ShapeDtypeStruct((GG, DD), jnp.float32),
    ],
)


def _last_body(p0_ref, p1_ref, hws_ref, dinv_ref, batch_ref, b_ref, pool_ref):
    i = pl.program_id(0)
    t = (p0_ref[0] + p1_ref[0] + hws_ref[...]) * dinv_ref[...]
    h = jnp.maximum(t + b_ref[...], 0.0)
    oneh = (batch_ref[...] == lax.broadcasted_iota(jnp.int32, (BR, GG), 1))
    oneh = oneh.astype(jnp.float32)
    contrib = lax.dot_general(
        oneh, h, (((0,), (0,)), ((), ())), preferred_element_type=jnp.float32
    )

    @pl.when(i == 0)
    def _():
        pool_ref[...] = jnp.zeros_like(pool_ref)

    pool_ref[...] += contrib


_last_call = pl.pallas_call(
    _last_body,
    grid=(NB,),
    in_specs=[
        pl.BlockSpec((1, BR, DD), lambda i: (0, i, 0)),
        pl.BlockSpec((1, BR, DD), lambda i: (1, i, 0)),
        pl.BlockSpec((BR, DD), lambda i: (i, 0)),
        pl.BlockSpec((BR, 1), lambda i: (i, 0)),
        pl.BlockSpec((BR, 1), lambda i: (i, 0)),
        pl.BlockSpec((1, DD), lambda i: (0, 0)),
    ],
    out_specs=pl.BlockSpec((GG, DD), lambda i: (0, 0)),
    out_shape=jax.ShapeDtypeStruct((GG, DD), jnp.float32),
)


def kernel(x, edge_index, batch, Ws, bs):
    edge_index = edge_index.astype(jnp.int32)
    npad = EP - EE
    src = jnp.concatenate([edge_index[0], jnp.zeros((npad,), jnp.int32)])
    dst = jnp.concatenate([edge_index[1], jnp.full((npad,), NN, jnp.int32)])
    batch2 = batch.astype(jnp.int32).reshape(NN, 1)

    degp = _deg_kernel(dst)
    hws, dinv = _first_call(x, Ws[0], degp, degp)

    pools = []
    for i in range(5):
        parts = _agg_kernel(hws, src, dst)
        if i < 4:
            hws, pool = _layer_call(
                parts, parts, hws, dinv, batch2,
                bs[i].reshape(1, DD), Ws[i + 1]
            )
        else:
            pool = _last_call(
                parts, parts, hws, dinv, batch2, bs[i].reshape(1, DD)
            )
        pools.append(pool)

    return jnp.concatenate(pools, axis=1)


# speedup vs baseline: 6.4629x; 1.0450x over previous
import functools
import jax
import jax.numpy as jnp
from jax import lax
from jax.experimental import pallas as pl
from jax.experimental.pallas import tpu as pltpu
from jax.experimental.pallas import tpu_sc as plsc

NN = 10000
EE = 320000
DD = 128
GG = 64
NC = 2
NS = 16
NW = NC * NS
CH = 128
NNP = 10240
RPT = NNP // NS
ZR = RPT // 5
EPW = 10240
EP = NW * EPW
NCHUNK = EPW // CH
FAST_CID = 1
NCF = 124
NCS = 36
EPWF = NCF * CH
EPWS = NCS * CH
BR = 1000
NB = NN // BR

_mesh = plsc.VectorSubcoreMesh(core_axis_name="c", subcore_axis_name="s")


def _zero_acc(zero_v, acc, sid, sem):
    @pl.loop(0, ZR)
    def _z(i):
        for k in range(DD // 16):
            zero_v[i, pl.ds(k * 16, 16)] = jnp.zeros((16,), jnp.float32)

    for k in range(5):
        pltpu.async_copy(
            zero_v.at[pl.ds(0, ZR), :], acc.at[pl.ds(sid * RPT + k * ZR, ZR), :], sem
        )
    for k in range(5):
        pltpu.make_async_copy(
            zero_v.at[pl.ds(0, ZR), :], acc.at[pl.ds(sid * RPT, ZR), :], sem
        ).wait()


@functools.partial(
    pl.kernel,
    out_type=jax.ShapeDtypeStruct((NC, NNP, DD), jnp.float32),
    mesh=_mesh,
    scratch_types=[
        [pltpu.VMEM((CH,), jnp.int32) for _ in range(4)],
        pltpu.VMEM((CH, DD), jnp.float32),
        pltpu.VMEM_SHARED((NNP, DD), jnp.float32),
        pltpu.SemaphoreType.DMA,
        pltpu.SemaphoreType.DMA,
    ],
)
def _deg_kernel(dst_hbm, out_hbm, dbufs, ones_v, acc, isem, ssem):
    cid = lax.axis_index("c")
    sid = lax.axis_index("s")
    wid = sid * NC + cid
    ebase = wid * EPW

    def start_i(j, buf):
        pltpu.async_copy(dst_hbm.at[pl.ds(ebase + j * CH, CH)], buf, isem)

    def wait_i(buf):
        pltpu.make_async_copy(dst_hbm.at[pl.ds(ebase, CH)], buf, isem).wait()

    def wait_s():
        pltpu.make_async_copy(ones_v, acc.at[dbufs[0]], ssem).wait()

    for m in range(3):
        start_i(m, dbufs[m])
    _zero_acc(ones_v, acc, sid, ssem)

    @pl.loop(0, CH)
    def _o(i):
        for k in range(DD // 16):
            ones_v[i, pl.ds(k * 16, 16)] = jnp.full((16,), 1.0, jnp.float32)

    wait_i(dbufs[0])
    plsc.subcore_barrier()

    Q = NCHUNK // 4

    @pl.loop(0, Q)
    def _chunk(q):
        for m in range(4):
            j = q * 4 + m
            pltpu.async_copy(ones_v, acc.at[dbufs[m]], ssem, add=True)
            if m == 0:
                @pl.when(q > 0)
                def _():
                    wait_s()
            else:
                wait_s()
            if m == 1 or m == 2 or m == 3:
                @pl.when(q < Q - 1)
                def _():
                    start_i(j + 3, dbufs[(m + 3) % 4])
            else:
                start_i(j + 3, dbufs[3])
            if m < 3:
                wait_i(dbufs[m + 1])
            else:
                @pl.when(q < Q - 1)
                def _():
                    wait_i(dbufs[0])

    wait_s()

    plsc.subcore_barrier()
    pltpu.async_copy(
        acc.at[pl.ds(sid * RPT, RPT), :],
        out_hbm.at[cid, pl.ds(sid * RPT, RPT), :],
        ssem,
    ).wait()


@functools.partial(
    pl.kernel,
    out_type=jax.ShapeDtypeStruct((NC, NNP, DD), jnp.float32),
    mesh=_mesh,
    scratch_types=[
        [pltpu.VMEM((CH,), jnp.int32) for _ in range(4)],
        [pltpu.VMEM((CH,), jnp.int32) for _ in range(4)],
        pltpu.VMEM((CH, DD), jnp.float32),
        pltpu.VMEM((CH, DD), jnp.float32),
        pltpu.VMEM_SHARED((NNP, DD), jnp.float32),
        pltpu.SemaphoreType.DMA,
        pltpu.SemaphoreType.DMA,
        pltpu.SemaphoreType.DMA,
    ],
)
def _agg_kernel(hws_hbm, src_hbm, dst_hbm, out_hbm,
                sbufs, dbufs, rows0, rows1, acc,
                isem, gsem, ssem):
    cid = lax.axis_index("c")
    sid = lax.axis_index("s")
    rbufs = (rows0, rows1)
    is_fast = cid == FAST_CID
    ebase = jnp.where(is_fast, sid * EPWF, NS * EPWF + sid * EPWS)
    Q = jnp.where(is_fast, NCF // 4, NCS // 4)

    def start_i(j, m):
        pltpu.async_copy(src_hbm.at[pl.ds(ebase + j * CH, CH)], sbufs[m], isem)
        pltpu.async_copy(dst_hbm.at[pl.ds(ebase + j * CH, CH)], dbufs[m], isem)

    def wait_i(m):
        pltpu.make_async_copy(src_hbm.at[pl.ds(ebase, CH)], sbufs[m], isem).wait()
        pltpu.make_async_copy(dst_hbm.at[pl.ds(ebase, CH)], dbufs[m], isem).wait()

    def start_g(m, buf):
        pltpu.async_copy(hws_hbm.at[sbufs[m]], buf, gsem)

    def wait_g(buf):
        pltpu.make_async_copy(hws_hbm.at[sbufs[0]], buf, gsem).wait()

    def start_s(m, buf):
        pltpu.async_copy(buf, acc.at[dbufs[m]], ssem, add=True)

    def wait_s(buf):
        pltpu.make_async_copy(buf, acc.at[dbufs[0]], ssem).wait()

    for m in range(3):
        start_i(m, m)
    _zero_acc(rows0, acc, sid, gsem)
    wait_i(0)
    plsc.subcore_barrier()
    start_g(0, rows0)

    @pl.loop(0, Q)
    def _chunk(q):
        for m in range(4):
            j = q * 4 + m
            cur = rbufs[m % 2]
            oth = rbufs[1 - m % 2]

            wait_g(cur)
            start_s(m, cur)
            if m == 0:
                @pl.when(q > 0)
                def _():
                    wait_s(oth)
            else:
                wait_s(oth)
            if m < 3:
                wait_i(m + 1)
                start_g(m + 1, oth)
            else:
                @pl.when(q < Q - 1)
                def _():
                    wait_i(0)
                    start_g(0, oth)
            if m == 0:
                start_i(j + 3, 3)
            else:
                @pl.when(q < Q - 1)
                def _():
                    start_i(j + 3, (m + 3) % 4)

    wait_s(rows0)

    plsc.subcore_barrier()
    pltpu.async_copy(
        acc.at[pl.ds(sid * RPT, RPT), :],
        out_hbm.at[cid, pl.ds(sid * RPT, RPT), :],
        ssem,
    ).wait()


def _first_body(x_ref, w_ref, d0_ref, d1_ref, hws_ref, dinv_ref):
    deg = d0_ref[0, :, 0:1] + d1_ref[0, :, 0:1] + 1.0
    dinv = lax.rsqrt(deg)
    dinv_ref[...] = dinv
    hws_ref[...] = (
        jnp.dot(x_ref[...], w_ref[...], preferred_element_type=jnp.float32) * dinv
    )


_first_call = pl.pallas_call(
    _first_body,
    grid=(NB,),
    in_specs=[
        pl.BlockSpec((BR, DD), lambda i: (i, 0)),
        pl.BlockSpec((DD, DD), lambda i: (0, 0)),
        pl.BlockSpec((1, BR, DD), lambda i: (0, i, 0)),
        pl.BlockSpec((1, BR, DD), lambda i: (1, i, 0)),
    ],
    out_specs=[
        pl.BlockSpec((BR, DD), lambda i: (i, 0)),
        pl.BlockSpec((BR, 1), lambda i: (i, 0)),
    ],
    out_shape=[
        jax.ShapeDtypeStruct((NN, DD), jnp.float32),
        jax.ShapeDtypeStruct((NN, 1), jnp.float32),
    ],
)


def _layer_body(p0_ref, p1_ref, hws_ref, dinv_ref, batch_ref, b_ref, w_ref,
                hwsn_ref, pool_ref):
    i = pl.program_id(0)
    dinv = dinv_ref[...]
    t = (p0_ref[0] + p1_ref[0] + hws_ref[...]) * dinv
    h = jnp.maximum(t + b_ref[...], 0.0)
    oneh = (batch_ref[...] == lax.broadcasted_iota(jnp.int32, (BR, GG), 1))
    oneh = oneh.astype(jnp.float32)
    contrib = lax.dot_general(
        oneh, h, (((0,), (0,)), ((), ())), preferred_element_type=jnp.float32
    )

    @pl.when(i == 0)
    def _():
        pool_ref[...] = jnp.zeros_like(pool_ref)

    pool_ref[...] += contrib
    hwsn_ref[...] = (
        jnp.dot(h, w_ref[...], preferred_element_type=jnp.float32) * dinv
    )


_layer_call = pl.pallas_call(
    _layer_body,
    grid=(NB,),
    in_specs=[
        pl.BlockSpec((1, BR, DD), lambda i: (0, i, 0)),
        pl.BlockSpec((1, BR, DD), lambda i: (1, i, 0)),
        pl.BlockSpec((BR, DD), lambda i: (i, 0)),
        pl.BlockSpec((BR, 1), lambda i: (i, 0)),
        pl.BlockSpec((BR, 1), lambda i: (i, 0)),
        pl.BlockSpec((1, DD), lambda i: (0, 0)),
        pl.BlockSpec((DD, DD), lambda i: (0, 0)),
    ],
    out_specs=[
        pl.BlockSpec((BR, DD), lambda i: (i, 0)),
        pl.BlockSpec((GG, DD), lambda i: (0, 0)),
    ],
    out_shape=[
        jax.ShapeDtypeStruct((NN, DD), jnp.float32),
        jax.ShapeDtypeStruct((GG, DD), jnp.float32),
    ],
)


def _last_body(p0_ref, p1_ref, hws_ref, dinv_ref, batch_ref, b_ref, pool_ref):
    i = pl.program_id(0)
    t = (p0_ref[0] + p1_ref[0] + hws_ref[...]) * dinv_ref[...]
    h = jnp.maximum(t + b_ref[...], 0.0)
    oneh = (batch_ref[...] == lax.broadcasted_iota(jnp.int32, (BR, GG), 1))
    oneh = oneh.astype(jnp.float32)
    contrib = lax.dot_general(
        oneh, h, (((0,), (0,)), ((), ())), preferred_element_type=jnp.float32
    )

    @pl.when(i == 0)
    def _():
        pool_ref[...] = jnp.zeros_like(pool_ref)

    pool_ref[...] += contrib


_last_call = pl.pallas_call(
    _last_body,
    grid=(NB,),
    in_specs=[
        pl.BlockSpec((1, BR, DD), lambda i: (0, i, 0)),
        pl.BlockSpec((1, BR, DD), lambda i: (1, i, 0)),
        pl.BlockSpec((BR, DD), lambda i: (i, 0)),
        pl.BlockSpec((BR, 1), lambda i: (i, 0)),
        pl.BlockSpec((BR, 1), lambda i: (i, 0)),
        pl.BlockSpec((1, DD), lambda i: (0, 0)),
    ],
    out_specs=pl.BlockSpec((GG, DD), lambda i: (0, 0)),
    out_shape=jax.ShapeDtypeStruct((GG, DD), jnp.float32),
)


def kernel(x, edge_index, batch, Ws, bs):
    edge_index = edge_index.astype(jnp.int32)
    npad = EP - EE
    src = jnp.concatenate([edge_index[0], jnp.zeros((npad,), jnp.int32)])
    dst = jnp.concatenate([edge_index[1], jnp.full((npad,), NN, jnp.int32)])
    batch2 = batch.astype(jnp.int32).reshape(NN, 1)

    degp = _deg_kernel(dst)
    hws, dinv = _first_call(x, Ws[0], degp, degp)

    pools = []
    for i in range(5):
        parts = _agg_kernel(hws, src, dst)
        if i < 4:
            hws, pool = _layer_call(
                parts, parts, hws, dinv, batch2,
                bs[i].reshape(1, DD), Ws[i + 1]
            )
        else:
            pool = _last_call(
                parts, parts, hws, dinv, batch2, bs[i].reshape(1, DD)
            )
        pools.append(pool)

    return jnp.concatenate(pools, axis=1)
